# Initial kernel scaffold; baseline (speedup 1.0000x reference)
import jax
import jax.numpy as jnp
from jax.experimental import pallas as pl


def kernel(x, weight):
    raise NotImplementedError("write your pallas kernel here")


# speedup vs baseline: 1.7987x; 1.7987x over previous
import functools

import jax
import jax.numpy as jnp
from jax import lax
from jax.experimental import pallas as pl
from jax.experimental.pallas import tpu as pltpu
from jax.experimental.pallas import tpu_sc as plsc

_NUM_WORKERS = 32
_CHUNK = 512


@functools.cache
def _build(n_rows, vocab, dim, chunk):
    n_per_w = n_rows // _NUM_WORKERS
    n_chunks = n_per_w // chunk
    mesh = plsc.VectorSubcoreMesh(core_axis_name="c", subcore_axis_name="s")

    @functools.partial(
        pl.kernel,
        mesh=mesh,
        out_type=jax.ShapeDtypeStruct((n_rows, dim), jnp.float32),
        scratch_types=[
            pltpu.VMEM((chunk,), jnp.int32),
            pltpu.VMEM((chunk, dim), jnp.float32),
            pltpu.SemaphoreType.DMA,
        ],
        compiler_params=pltpu.CompilerParams(use_tc_tiling_on_sc=False),
    )
    def emb(x_hbm, w_hbm, out_hbm, idx_v, rows_v, sem):
        wid = lax.axis_index("s") * 2 + lax.axis_index("c")
        base = wid * n_per_w

        def body(g, carry):
            off = base + g * chunk
            pltpu.sync_copy(x_hbm.at[pl.ds(off, chunk)], idx_v)
            pltpu.async_copy(w_hbm.at[idx_v], rows_v, sem).wait()
            pltpu.sync_copy(rows_v, out_hbm.at[pl.ds(off, chunk)])
            return carry

        lax.fori_loop(0, n_chunks, body, 0)

    return emb


def kernel(x, weight):
    b, s = x.shape
    vocab, dim = weight.shape
    xf = x.reshape(-1).astype(jnp.int32)
    emb = _build(b * s, vocab, dim, _CHUNK)
    out = emb(xf, weight)
    return out.reshape(b, s, dim)


# speedup vs baseline: 1.8887x; 1.0501x over previous
import functools

import jax
import jax.numpy as jnp
from jax import lax
from jax.experimental import pallas as pl
from jax.experimental.pallas import tpu as pltpu
from jax.experimental.pallas import tpu_sc as plsc

_NUM_WORKERS = 32
_CHUNK = 512
_NBUF = 2


@functools.cache
def _build(n_rows, vocab, dim, chunk):
    n_per_w = n_rows // _NUM_WORKERS
    n_chunks = n_per_w // chunk
    n_steady = n_chunks - _NBUF
    assert n_steady % _NBUF == 0 and n_steady >= 0
    mesh = plsc.VectorSubcoreMesh(core_axis_name="c", subcore_axis_name="s")

    @functools.partial(
        pl.kernel,
        mesh=mesh,
        out_type=jax.ShapeDtypeStruct((n_rows, dim), jnp.float32),
        scratch_types=[
            pltpu.VMEM((_NBUF, chunk), jnp.int32),
            pltpu.VMEM((_NBUF, chunk, dim), jnp.float32),
            [pltpu.SemaphoreType.DMA] * _NBUF,
            [pltpu.SemaphoreType.DMA] * _NBUF,
        ],
        compiler_params=pltpu.CompilerParams(use_tc_tiling_on_sc=False),
    )
    def emb(x_hbm, w_hbm, out_hbm, idx_v, rows_v, sem_g, sem_s):
        wid = lax.axis_index("s") * 2 + lax.axis_index("c")
        base = wid * n_per_w

        for b in range(_NBUF):
            off = base + b * chunk
            pltpu.sync_copy(x_hbm.at[pl.ds(off, chunk)], idx_v.at[b])
            pltpu.async_copy(w_hbm.at[idx_v.at[b]], rows_v.at[b], sem_g[b])

        def body(p, carry):
            for b in range(_NBUF):
                c = p * _NBUF + b
                off = base + c * chunk
                pltpu.make_async_copy(
                    w_hbm.at[idx_v.at[b]], rows_v.at[b], sem_g[b]
                ).wait()
                pltpu.async_copy(
                    rows_v.at[b], out_hbm.at[pl.ds(off, chunk)], sem_s[b]
                )
                off2 = off + _NBUF * chunk
                pltpu.sync_copy(x_hbm.at[pl.ds(off2, chunk)], idx_v.at[b])
                pltpu.make_async_copy(
                    rows_v.at[b], out_hbm.at[pl.ds(off, chunk)], sem_s[b]
                ).wait()
                pltpu.async_copy(w_hbm.at[idx_v.at[b]], rows_v.at[b], sem_g[b])
            return carry

        lax.fori_loop(0, n_steady // _NBUF, body, 0)

        for b in range(_NBUF):
            c = n_steady + b
            off = base + c * chunk
            pltpu.make_async_copy(
                w_hbm.at[idx_v.at[b]], rows_v.at[b], sem_g[b]
            ).wait()
            pltpu.async_copy(
                rows_v.at[b], out_hbm.at[pl.ds(off, chunk)], sem_s[b]
            )
        for b in range(_NBUF):
            c = n_steady + b
            off = base + c * chunk
            pltpu.make_async_copy(
                rows_v.at[b], out_hbm.at[pl.ds(off, chunk)], sem_s[b]
            ).wait()

    return emb


def kernel(x, weight):
    b, s = x.shape
    vocab, dim = weight.shape
    xf = x.reshape(-1).astype(jnp.int32)
    emb = _build(b * s, vocab, dim, _CHUNK)
    out = emb(xf, weight)
    return out.reshape(b, s, dim)
